# Initial kernel scaffold; baseline (speedup 1.0000x reference)
import functools

import jax
import jax.numpy as jnp
from jax import lax
from jax.experimental import pallas as pl
from jax.experimental.pallas import tpu as pltpu
from jax.experimental.pallas import tpu_sc as plsc

N_NODES = 10000
N_EDGES = 160000
D_IN = 256
D_H1 = 256
D_H2 = 128

NC = 2
NS = 16
CHUNK = 80
EDGES_PER_SUB = N_EDGES // NS
N_CHUNKS = EDGES_PER_SUB // CHUNK
ROWS_PER_SUB = N_NODES // NS


def _make_agg(dc):
  mesh = plsc.VectorSubcoreMesh(core_axis_name="c", subcore_axis_name="s")
  out_type = (
      jax.ShapeDtypeStruct((N_NODES, dc), jnp.float32),
      jax.ShapeDtypeStruct((N_NODES, dc), jnp.float32),
  )
  scratch = [
      pltpu.VMEM((CHUNK,), jnp.int32),
      pltpu.VMEM((CHUNK,), jnp.int32),
      pltpu.VMEM((CHUNK, dc), jnp.float32),
      pltpu.VMEM_SHARED((N_NODES, dc), jnp.float32),
      pltpu.SemaphoreType.DMA,
  ]

  @functools.partial(pl.kernel, out_type=out_type, mesh=mesh,
                     scratch_types=scratch)
  def agg(t0_hbm, t1_hbm, src_hbm, dst_hbm, zeros_hbm, out0_hbm, out1_hbm,
          src_v, dst_v, rows_v, acc_sh, sem):
    cid = lax.axis_index("c")
    sid = lax.axis_index("s")
    row0 = sid * ROWS_PER_SUB

    pltpu.sync_copy(zeros_hbm.at[pl.ds(row0, ROWS_PER_SUB)],
                    acc_sh.at[pl.ds(row0, ROWS_PER_SUB)])
    plsc.subcore_barrier()

    def edges(table_hbm):
      def step(i, carry):
        base = sid * EDGES_PER_SUB + i * CHUNK
        pltpu.sync_copy(src_hbm.at[pl.ds(base, CHUNK)], src_v)
        pltpu.sync_copy(dst_hbm.at[pl.ds(base, CHUNK)], dst_v)
        pltpu.async_copy(table_hbm.at[src_v], rows_v, sem).wait()
        pltpu.sync_copy(rows_v, acc_sh.at[dst_v], add=True)
        return carry
      lax.fori_loop(0, N_CHUNKS, step, 0)

    @pl.when(cid == 0)
    def _():
      edges(t0_hbm)

    @pl.when(cid == 1)
    def _():
      edges(t1_hbm)

    plsc.subcore_barrier()

    @pl.when(cid == 0)
    def _():
      pltpu.sync_copy(acc_sh.at[pl.ds(row0, ROWS_PER_SUB)],
                      out0_hbm.at[pl.ds(row0, ROWS_PER_SUB)])

    @pl.when(cid == 1)
    def _():
      pltpu.sync_copy(acc_sh.at[pl.ds(row0, ROWS_PER_SUB)],
                      out1_hbm.at[pl.ds(row0, ROWS_PER_SUB)])

  return agg


_agg_l1 = _make_agg(D_IN // 2)
_agg_l2 = _make_agg(D_H2 // 2)


BM = 1000


def _mm_body(a0_ref, a1_ref, w1_ref, w2_ref, o0_ref, o1_ref):
  h = jnp.dot(a0_ref[...], w1_ref[0:128, :], preferred_element_type=jnp.float32)
  h = h + jnp.dot(a1_ref[...], w1_ref[128:256, :],
                  preferred_element_type=jnp.float32)
  h1 = jnp.where(h > 0, h, jnp.expm1(h))
  s2 = jnp.dot(h1, w2_ref[...], preferred_element_type=jnp.float32)
  o0_ref[...] = s2[:, 0:64]
  o1_ref[...] = s2[:, 64:128]


def _fused_mm(a0, a1, w1, w2):
  grid = (N_NODES // BM,)
  return pl.pallas_call(
      _mm_body,
      grid=grid,
      in_specs=[
          pl.BlockSpec((BM, 128), lambda i: (i, 0)),
          pl.BlockSpec((BM, 128), lambda i: (i, 0)),
          pl.BlockSpec((256, 256), lambda i: (0, 0)),
          pl.BlockSpec((256, 128), lambda i: (0, 0)),
      ],
      out_specs=[
          pl.BlockSpec((BM, 64), lambda i: (i, 0)),
          pl.BlockSpec((BM, 64), lambda i: (i, 0)),
      ],
      out_shape=[
          jax.ShapeDtypeStruct((N_NODES, 64), jnp.float32),
          jax.ShapeDtypeStruct((N_NODES, 64), jnp.float32),
      ],
  )(a0, a1, w1, w2)


def _norm_body(h0_ref, h1_ref, o_ref):
  a = h0_ref[...]
  b = h1_ref[...]
  ss = (jnp.sum(a * a, axis=1, keepdims=True)
        + jnp.sum(b * b, axis=1, keepdims=True))
  inv = 1.0 / jnp.maximum(jnp.sqrt(ss), 1e-12)
  o_ref[...] = jnp.concatenate([a * inv, b * inv], axis=1)


def _normalize(h0, h1):
  grid = (N_NODES // BM,)
  return pl.pallas_call(
      _norm_body,
      grid=grid,
      in_specs=[
          pl.BlockSpec((BM, 64), lambda i: (i, 0)),
          pl.BlockSpec((BM, 64), lambda i: (i, 0)),
      ],
      out_specs=pl.BlockSpec((BM, 128), lambda i: (i, 0)),
      out_shape=jax.ShapeDtypeStruct((N_NODES, D_H2), jnp.float32),
  )(h0, h1)


@jax.jit
def kernel(x, edge_index, W1, W2):
  src = edge_index[0].astype(jnp.int32)
  dst = edge_index[1].astype(jnp.int32)
  x0 = x[:, :128]
  x1 = x[:, 128:]
  z128 = jnp.zeros((N_NODES, 128), jnp.float32)
  z64 = jnp.zeros((N_NODES, 64), jnp.float32)

  agg0, agg1 = _agg_l1(x0, x1, src, dst, z128)
  s20, s21 = _fused_mm(agg0, agg1, W1, W2)
  h20, h21 = _agg_l2(s20, s21, src, dst, z64)
  return _normalize(h20, h21)


# speedup vs baseline: 4.0374x; 4.0374x over previous
import functools

import jax
import jax.numpy as jnp
from jax import lax
from jax.experimental import pallas as pl
from jax.experimental.pallas import tpu as pltpu
from jax.experimental.pallas import tpu_sc as plsc

N_NODES = 10000
N_EDGES = 160000
D_IN = 256
D_H1 = 256
D_H2 = 128

NC = 2
NS = 16
CHUNK = 80
EDGES_PER_SUB = N_EDGES // NS
N_CHUNKS = EDGES_PER_SUB // CHUNK
SLAB = 632
SLAB_LAST = N_NODES - (NS - 1) * SLAB


def _make_agg(dc):
  mesh = plsc.VectorSubcoreMesh(core_axis_name="c", subcore_axis_name="s")
  out_type = (
      jax.ShapeDtypeStruct((N_NODES, dc), jnp.float32),
      jax.ShapeDtypeStruct((N_NODES, dc), jnp.float32),
  )
  scratch = [
      pltpu.VMEM((CHUNK,), jnp.int32),
      pltpu.VMEM((CHUNK,), jnp.int32),
      pltpu.VMEM((CHUNK, dc), jnp.float32),
      pltpu.VMEM_SHARED((N_NODES, dc), jnp.float32),
      pltpu.SemaphoreType.DMA,
  ]

  @functools.partial(pl.kernel, out_type=out_type, mesh=mesh,
                     scratch_types=scratch)
  def agg(t0_hbm, t1_hbm, src_hbm, dst_hbm, zeros_hbm, out0_hbm, out1_hbm,
          src_v, dst_v, rows_v, acc_sh, sem):
    cid = lax.axis_index("c")
    sid = lax.axis_index("s")
    row0 = sid * SLAB

    def slab_copy(make_src, make_dst):
      @pl.when(sid < NS - 1)
      def _():
        pltpu.sync_copy(make_src(row0, SLAB), make_dst(row0, SLAB))

      @pl.when(sid == NS - 1)
      def _():
        pltpu.sync_copy(make_src(row0, SLAB_LAST), make_dst(row0, SLAB_LAST))

    slab_copy(lambda r, n: zeros_hbm.at[pl.ds(r, n)],
              lambda r, n: acc_sh.at[pl.ds(r, n)])
    plsc.subcore_barrier()

    def edges(table_hbm):
      def step(i, carry):
        base = sid * EDGES_PER_SUB + i * CHUNK
        pltpu.sync_copy(src_hbm.at[pl.ds(base, CHUNK)], src_v)
        pltpu.sync_copy(dst_hbm.at[pl.ds(base, CHUNK)], dst_v)
        pltpu.async_copy(table_hbm.at[src_v], rows_v, sem).wait()
        pltpu.sync_copy(rows_v, acc_sh.at[dst_v], add=True)
        return carry
      lax.fori_loop(0, N_CHUNKS, step, 0)

    @pl.when(cid == 0)
    def _():
      edges(t0_hbm)

    @pl.when(cid == 1)
    def _():
      edges(t1_hbm)

    plsc.subcore_barrier()

    @pl.when(cid == 0)
    def _():
      slab_copy(lambda r, n: acc_sh.at[pl.ds(r, n)],
                lambda r, n: out0_hbm.at[pl.ds(r, n)])

    @pl.when(cid == 1)
    def _():
      slab_copy(lambda r, n: acc_sh.at[pl.ds(r, n)],
                lambda r, n: out1_hbm.at[pl.ds(r, n)])

  return agg


_agg_l1 = _make_agg(D_IN // 2)


def _make_agg_edgesplit(dc):
  epc = N_EDGES // NC
  eps = epc // NS
  n_full = eps // CHUNK
  tail = eps - n_full * CHUNK
  mesh = plsc.VectorSubcoreMesh(core_axis_name="c", subcore_axis_name="s")
  out_type = (
      jax.ShapeDtypeStruct((N_NODES, dc), jnp.float32),
      jax.ShapeDtypeStruct((N_NODES, dc), jnp.float32),
  )
  scratch = [
      pltpu.VMEM((CHUNK,), jnp.int32),
      pltpu.VMEM((CHUNK,), jnp.int32),
      pltpu.VMEM((CHUNK, dc), jnp.float32),
      pltpu.VMEM((tail,), jnp.int32),
      pltpu.VMEM((tail,), jnp.int32),
      pltpu.VMEM((tail, dc), jnp.float32),
      pltpu.VMEM_SHARED((N_NODES, dc), jnp.float32),
      pltpu.SemaphoreType.DMA,
  ]

  @functools.partial(pl.kernel, out_type=out_type, mesh=mesh,
                     scratch_types=scratch)
  def agg(table_hbm, src_hbm, dst_hbm, zeros_hbm, out0_hbm, out1_hbm,
          src_v, dst_v, rows_v, src_t, dst_t, rows_t, acc_sh, sem):
    cid = lax.axis_index("c")
    sid = lax.axis_index("s")
    row0 = sid * SLAB

    def slab_copy(make_src, make_dst):
      @pl.when(sid < NS - 1)
      def _():
        pltpu.sync_copy(make_src(row0, SLAB), make_dst(row0, SLAB))

      @pl.when(sid == NS - 1)
      def _():
        pltpu.sync_copy(make_src(row0, SLAB_LAST), make_dst(row0, SLAB_LAST))

    slab_copy(lambda r, n: zeros_hbm.at[pl.ds(r, n)],
              lambda r, n: acc_sh.at[pl.ds(r, n)])
    plsc.subcore_barrier()

    base0 = cid * epc + sid * eps

    def step(i, carry):
      base = base0 + i * CHUNK
      pltpu.sync_copy(src_hbm.at[pl.ds(base, CHUNK)], src_v)
      pltpu.sync_copy(dst_hbm.at[pl.ds(base, CHUNK)], dst_v)
      pltpu.async_copy(table_hbm.at[src_v], rows_v, sem).wait()
      pltpu.sync_copy(rows_v, acc_sh.at[dst_v], add=True)
      return carry
    lax.fori_loop(0, n_full, step, 0)

    if tail:
      base = base0 + n_full * CHUNK
      pltpu.sync_copy(src_hbm.at[pl.ds(base, tail)], src_t)
      pltpu.sync_copy(dst_hbm.at[pl.ds(base, tail)], dst_t)
      pltpu.async_copy(table_hbm.at[src_t], rows_t, sem).wait()
      pltpu.sync_copy(rows_t, acc_sh.at[dst_t], add=True)

    plsc.subcore_barrier()

    @pl.when(cid == 0)
    def _():
      slab_copy(lambda r, n: acc_sh.at[pl.ds(r, n)],
                lambda r, n: out0_hbm.at[pl.ds(r, n)])

    @pl.when(cid == 1)
    def _():
      slab_copy(lambda r, n: acc_sh.at[pl.ds(r, n)],
                lambda r, n: out1_hbm.at[pl.ds(r, n)])

  return agg


_agg_l2 = _make_agg_edgesplit(D_H2)


BM = 1000


def _mm_body(a0_ref, a1_ref, w1_ref, w2_ref, o_ref):
  h = jnp.dot(a0_ref[...], w1_ref[0:128, :], preferred_element_type=jnp.float32)
  h = h + jnp.dot(a1_ref[...], w1_ref[128:256, :],
                  preferred_element_type=jnp.float32)
  h1 = jnp.where(h > 0, h, jnp.exp(jnp.minimum(h, 0.0)) - 1.0)
  o_ref[...] = jnp.dot(h1, w2_ref[...], preferred_element_type=jnp.float32)


def _fused_mm(a0, a1, w1, w2):
  grid = (N_NODES // BM,)
  return pl.pallas_call(
      _mm_body,
      grid=grid,
      in_specs=[
          pl.BlockSpec((BM, 128), lambda i: (i, 0)),
          pl.BlockSpec((BM, 128), lambda i: (i, 0)),
          pl.BlockSpec((256, 256), lambda i: (0, 0)),
          pl.BlockSpec((256, 128), lambda i: (0, 0)),
      ],
      out_specs=pl.BlockSpec((BM, 128), lambda i: (i, 0)),
      out_shape=jax.ShapeDtypeStruct((N_NODES, D_H2), jnp.float32),
  )(a0, a1, w1, w2)


def _norm_body(p0_ref, p1_ref, o_ref):
  h2 = p0_ref[...] + p1_ref[...]
  ss = jnp.sum(h2 * h2, axis=1, keepdims=True)
  inv = 1.0 / jnp.maximum(jnp.sqrt(ss), 1e-12)
  o_ref[...] = h2 * inv


def _normalize(p0, p1):
  grid = (N_NODES // BM,)
  return pl.pallas_call(
      _norm_body,
      grid=grid,
      in_specs=[
          pl.BlockSpec((BM, 128), lambda i: (i, 0)),
          pl.BlockSpec((BM, 128), lambda i: (i, 0)),
      ],
      out_specs=pl.BlockSpec((BM, 128), lambda i: (i, 0)),
      out_shape=jax.ShapeDtypeStruct((N_NODES, D_H2), jnp.float32),
  )(p0, p1)


@jax.jit
def kernel(x, edge_index, W1, W2):
  src = edge_index[0].astype(jnp.int32)
  dst = edge_index[1].astype(jnp.int32)
  x0 = x[:, :128]
  x1 = x[:, 128:]
  z128 = jnp.zeros((N_NODES, 128), jnp.float32)

  agg0, agg1 = _agg_l1(x0, x1, src, dst, z128)
  s2 = _fused_mm(agg0, agg1, W1, W2)
  p0, p1 = _agg_l2(s2, src, dst, z128)
  return _normalize(p0, p1)


# speedup vs baseline: 5.4383x; 1.3470x over previous
import functools

import jax
import jax.numpy as jnp
from jax import lax
from jax.experimental import pallas as pl
from jax.experimental.pallas import tpu as pltpu
from jax.experimental.pallas import tpu_sc as plsc

N_NODES = 10000
N_EDGES = 160000
D_IN = 256
D_H1 = 256
D_H2 = 128

NC = 2
NS = 16
SLAB = 632
SLAB_LAST = N_NODES - (NS - 1) * SLAB


def _slab_copy(sid, make_src, make_dst):
  row0 = sid * SLAB

  @pl.when(sid < NS - 1)
  def _():
    pltpu.sync_copy(make_src(row0, SLAB), make_dst(row0, SLAB))

  @pl.when(sid == NS - 1)
  def _():
    pltpu.sync_copy(make_src(row0, SLAB_LAST), make_dst(row0, SLAB_LAST))


def _edge_pipeline(table_hbm, src_hbm, dst_hbm, acc_sh, base0, chunk, n_chunks,
                   src_a, dst_a, rows_a, sem_a, src_b, dst_b, rows_b, sem_b):
  def load(c, src_v, dst_v):
    b = base0 + c * chunk
    pltpu.sync_copy(src_hbm.at[pl.ds(b, chunk)], src_v)
    pltpu.sync_copy(dst_hbm.at[pl.ds(b, chunk)], dst_v)

  load(0, src_a, dst_a)
  pltpu.async_copy(table_hbm.at[src_a], rows_a, sem_a)

  def pair(j, carry):
    load(2 * j + 1, src_b, dst_b)
    pltpu.make_async_copy(table_hbm.at[src_a], rows_a, sem_a).wait()
    pltpu.async_copy(table_hbm.at[src_b], rows_b, sem_b)
    pltpu.sync_copy(rows_a, acc_sh.at[dst_a], add=True)
    load(2 * j + 2, src_a, dst_a)
    pltpu.make_async_copy(table_hbm.at[src_b], rows_b, sem_b).wait()
    pltpu.async_copy(table_hbm.at[src_a], rows_a, sem_a)
    pltpu.sync_copy(rows_b, acc_sh.at[dst_b], add=True)
    return carry

  lax.fori_loop(0, (n_chunks - 1) // 2, pair, 0)
  pltpu.make_async_copy(table_hbm.at[src_a], rows_a, sem_a).wait()
  pltpu.sync_copy(rows_a, acc_sh.at[dst_a], add=True)


def _gather_scratch(chunk, dc):
  return [
      pltpu.VMEM((chunk,), jnp.int32),
      pltpu.VMEM((chunk,), jnp.int32),
      pltpu.VMEM((chunk, dc), jnp.float32),
      pltpu.SemaphoreType.DMA,
      pltpu.VMEM((chunk,), jnp.int32),
      pltpu.VMEM((chunk,), jnp.int32),
      pltpu.VMEM((chunk, dc), jnp.float32),
      pltpu.SemaphoreType.DMA,
      pltpu.VMEM_SHARED((N_NODES, dc), jnp.float32),
  ]


CHUNK1 = 80
N_CHUNKS1 = N_EDGES // NS // CHUNK1


def _make_agg_colsplit(dc):
  mesh = plsc.VectorSubcoreMesh(core_axis_name="c", subcore_axis_name="s")
  out_type = (
      jax.ShapeDtypeStruct((N_NODES, dc), jnp.float32),
      jax.ShapeDtypeStruct((N_NODES, dc), jnp.float32),
  )

  @functools.partial(pl.kernel, out_type=out_type, mesh=mesh,
                     scratch_types=_gather_scratch(CHUNK1, dc))
  def agg(t0_hbm, t1_hbm, src_hbm, dst_hbm, zeros_hbm, out0_hbm, out1_hbm,
          src_a, dst_a, rows_a, sem_a, src_b, dst_b, rows_b, sem_b, acc_sh):
    cid = lax.axis_index("c")
    sid = lax.axis_index("s")

    _slab_copy(sid, lambda r, n: zeros_hbm.at[pl.ds(r, n)],
               lambda r, n: acc_sh.at[pl.ds(r, n)])
    plsc.subcore_barrier()

    base0 = sid * (N_EDGES // NS)

    @pl.when(cid == 0)
    def _():
      _edge_pipeline(t0_hbm, src_hbm, dst_hbm, acc_sh, base0, CHUNK1,
                     N_CHUNKS1, src_a, dst_a, rows_a, sem_a,
                     src_b, dst_b, rows_b, sem_b)

    @pl.when(cid == 1)
    def _():
      _edge_pipeline(t1_hbm, src_hbm, dst_hbm, acc_sh, base0, CHUNK1,
                     N_CHUNKS1, src_a, dst_a, rows_a, sem_a,
                     src_b, dst_b, rows_b, sem_b)

    plsc.subcore_barrier()

    @pl.when(cid == 0)
    def _():
      _slab_copy(sid, lambda r, n: acc_sh.at[pl.ds(r, n)],
                 lambda r, n: out0_hbm.at[pl.ds(r, n)])

    @pl.when(cid == 1)
    def _():
      _slab_copy(sid, lambda r, n: acc_sh.at[pl.ds(r, n)],
                 lambda r, n: out1_hbm.at[pl.ds(r, n)])

  return agg


_agg_l1 = _make_agg_colsplit(D_IN // 2)


CHUNK2 = 40
N_CHUNKS2 = N_EDGES // NC // NS // CHUNK2


def _make_agg_edgesplit(dc):
  epc = N_EDGES // NC
  eps = epc // NS
  mesh = plsc.VectorSubcoreMesh(core_axis_name="c", subcore_axis_name="s")
  out_type = (
      jax.ShapeDtypeStruct((N_NODES, dc), jnp.float32),
      jax.ShapeDtypeStruct((N_NODES, dc), jnp.float32),
  )

  @functools.partial(pl.kernel, out_type=out_type, mesh=mesh,
                     scratch_types=_gather_scratch(CHUNK2, dc))
  def agg(table_hbm, src_hbm, dst_hbm, zeros_hbm, out0_hbm, out1_hbm,
          src_a, dst_a, rows_a, sem_a, src_b, dst_b, rows_b, sem_b, acc_sh):
    cid = lax.axis_index("c")
    sid = lax.axis_index("s")

    _slab_copy(sid, lambda r, n: zeros_hbm.at[pl.ds(r, n)],
               lambda r, n: acc_sh.at[pl.ds(r, n)])
    plsc.subcore_barrier()

    base0 = cid * epc + sid * eps
    _edge_pipeline(table_hbm, src_hbm, dst_hbm, acc_sh, base0, CHUNK2,
                   N_CHUNKS2, src_a, dst_a, rows_a, sem_a,
                   src_b, dst_b, rows_b, sem_b)

    plsc.subcore_barrier()

    @pl.when(cid == 0)
    def _():
      _slab_copy(sid, lambda r, n: acc_sh.at[pl.ds(r, n)],
                 lambda r, n: out0_hbm.at[pl.ds(r, n)])

    @pl.when(cid == 1)
    def _():
      _slab_copy(sid, lambda r, n: acc_sh.at[pl.ds(r, n)],
                 lambda r, n: out1_hbm.at[pl.ds(r, n)])

  return agg


_agg_l2 = _make_agg_edgesplit(D_H2)


BM = 1000


def _mm_body(a0_ref, a1_ref, w1_ref, w2_ref, o_ref):
  h = jnp.dot(a0_ref[...], w1_ref[0:128, :], preferred_element_type=jnp.float32)
  h = h + jnp.dot(a1_ref[...], w1_ref[128:256, :],
                  preferred_element_type=jnp.float32)
  h1 = jnp.where(h > 0, h, jnp.exp(jnp.minimum(h, 0.0)) - 1.0)
  o_ref[...] = jnp.dot(h1, w2_ref[...], preferred_element_type=jnp.float32)


def _fused_mm(a0, a1, w1, w2):
  grid = (N_NODES // BM,)
  return pl.pallas_call(
      _mm_body,
      grid=grid,
      in_specs=[
          pl.BlockSpec((BM, 128), lambda i: (i, 0)),
          pl.BlockSpec((BM, 128), lambda i: (i, 0)),
          pl.BlockSpec((256, 256), lambda i: (0, 0)),
          pl.BlockSpec((256, 128), lambda i: (0, 0)),
      ],
      out_specs=pl.BlockSpec((BM, 128), lambda i: (i, 0)),
      out_shape=jax.ShapeDtypeStruct((N_NODES, D_H2), jnp.float32),
  )(a0, a1, w1, w2)


def _norm_body(p0_ref, p1_ref, o_ref):
  h2 = p0_ref[...] + p1_ref[...]
  ss = jnp.sum(h2 * h2, axis=1, keepdims=True)
  inv = 1.0 / jnp.maximum(jnp.sqrt(ss), 1e-12)
  o_ref[...] = h2 * inv


def _normalize(p0, p1):
  grid = (N_NODES // BM,)
  return pl.pallas_call(
      _norm_body,
      grid=grid,
      in_specs=[
          pl.BlockSpec((BM, 128), lambda i: (i, 0)),
          pl.BlockSpec((BM, 128), lambda i: (i, 0)),
      ],
      out_specs=pl.BlockSpec((BM, 128), lambda i: (i, 0)),
      out_shape=jax.ShapeDtypeStruct((N_NODES, D_H2), jnp.float32),
  )(p0, p1)


@jax.jit
def kernel(x, edge_index, W1, W2):
  src = edge_index[0].astype(jnp.int32)
  dst = edge_index[1].astype(jnp.int32)
  x0 = x[:, :128]
  x1 = x[:, 128:]
  z128 = jnp.zeros((N_NODES, 128), jnp.float32)

  agg0, agg1 = _agg_l1(x0, x1, src, dst, z128)
  s2 = _fused_mm(agg0, agg1, W1, W2)
  p0, p1 = _agg_l2(s2, src, dst, z128)
  return _normalize(p0, p1)


# speedup vs baseline: 7.4179x; 1.3640x over previous
import functools

import jax
import jax.numpy as jnp
from jax import lax
from jax.experimental import pallas as pl
from jax.experimental.pallas import tpu as pltpu
from jax.experimental.pallas import tpu_sc as plsc

N_NODES = 10000
N_EDGES = 160000
D_IN = 256
D_H1 = 256
D_H2 = 128

NC = 2
NS = 16
SLAB = 632
SLAB_LAST = N_NODES - (NS - 1) * SLAB


def _slab_copy(sid, make_src, make_dst):
  row0 = sid * SLAB

  @pl.when(sid < NS - 1)
  def _():
    pltpu.sync_copy(make_src(row0, SLAB), make_dst(row0, SLAB))

  @pl.when(sid == NS - 1)
  def _():
    pltpu.sync_copy(make_src(row0, SLAB_LAST), make_dst(row0, SLAB_LAST))


def _edge_pipeline(table_hbm, src_hbm, dst_hbm, acc_sh, base0, chunk, n_chunks,
                   tail, src_a, dst_a, rows_a, sem_a, src_b, dst_b, rows_b,
                   sem_b, src_t, dst_t, rows_t, sem_t):
  def load(c, src_v, dst_v):
    b = base0 + c * chunk
    pltpu.sync_copy(src_hbm.at[pl.ds(b, chunk)], src_v)
    pltpu.sync_copy(dst_hbm.at[pl.ds(b, chunk)], dst_v)

  if tail:
    bt = base0 + n_chunks * chunk
    pltpu.sync_copy(src_hbm.at[pl.ds(bt, tail)], src_t)
    pltpu.sync_copy(dst_hbm.at[pl.ds(bt, tail)], dst_t)
    pltpu.async_copy(table_hbm.at[src_t], rows_t, sem_t)

  load(0, src_a, dst_a)
  pltpu.async_copy(table_hbm.at[src_a], rows_a, sem_a)

  def pair(j, carry):
    load(2 * j + 1, src_b, dst_b)
    pltpu.make_async_copy(table_hbm.at[src_a], rows_a, sem_a).wait()
    pltpu.async_copy(table_hbm.at[src_b], rows_b, sem_b)
    pltpu.sync_copy(rows_a, acc_sh.at[dst_a], add=True)
    load(2 * j + 2, src_a, dst_a)
    pltpu.make_async_copy(table_hbm.at[src_b], rows_b, sem_b).wait()
    pltpu.async_copy(table_hbm.at[src_a], rows_a, sem_a)
    pltpu.sync_copy(rows_b, acc_sh.at[dst_b], add=True)
    return carry

  if n_chunks % 2:
    lax.fori_loop(0, (n_chunks - 1) // 2, pair, 0)
    pltpu.make_async_copy(table_hbm.at[src_a], rows_a, sem_a).wait()
    pltpu.sync_copy(rows_a, acc_sh.at[dst_a], add=True)
  else:
    lax.fori_loop(0, (n_chunks - 2) // 2, pair, 0)
    load(n_chunks - 1, src_b, dst_b)
    pltpu.make_async_copy(table_hbm.at[src_a], rows_a, sem_a).wait()
    pltpu.async_copy(table_hbm.at[src_b], rows_b, sem_b)
    pltpu.sync_copy(rows_a, acc_sh.at[dst_a], add=True)
    pltpu.make_async_copy(table_hbm.at[src_b], rows_b, sem_b).wait()
    pltpu.sync_copy(rows_b, acc_sh.at[dst_b], add=True)

  if tail:
    pltpu.make_async_copy(table_hbm.at[src_t], rows_t, sem_t).wait()
    pltpu.sync_copy(rows_t, acc_sh.at[dst_t], add=True)


def _gather_scratch(chunk, tail, dc):
  return [
      pltpu.VMEM((chunk,), jnp.int32),
      pltpu.VMEM((chunk,), jnp.int32),
      pltpu.VMEM((chunk, dc), jnp.float32),
      pltpu.SemaphoreType.DMA,
      pltpu.VMEM((chunk,), jnp.int32),
      pltpu.VMEM((chunk,), jnp.int32),
      pltpu.VMEM((chunk, dc), jnp.float32),
      pltpu.SemaphoreType.DMA,
      pltpu.VMEM((max(tail, 8),), jnp.int32),
      pltpu.VMEM((max(tail, 8),), jnp.int32),
      pltpu.VMEM((max(tail, 8), dc), jnp.float32),
      pltpu.SemaphoreType.DMA,
      pltpu.VMEM_SHARED((N_NODES, dc), jnp.float32),
  ]


CHUNK1 = 128
N_CHUNKS1 = N_EDGES // NS // CHUNK1
TAIL1 = N_EDGES // NS - N_CHUNKS1 * CHUNK1


def _make_agg_colsplit(dc):
  mesh = plsc.VectorSubcoreMesh(core_axis_name="c", subcore_axis_name="s")
  out_type = (
      jax.ShapeDtypeStruct((N_NODES, dc), jnp.float32),
      jax.ShapeDtypeStruct((N_NODES, dc), jnp.float32),
  )

  @functools.partial(pl.kernel, out_type=out_type, mesh=mesh,
                     scratch_types=_gather_scratch(CHUNK1, TAIL1, dc))
  def agg(t0_hbm, t1_hbm, src_hbm, dst_hbm, zeros_hbm, out0_hbm, out1_hbm,
          src_a, dst_a, rows_a, sem_a, src_b, dst_b, rows_b, sem_b,
          src_t, dst_t, rows_t, sem_t, acc_sh):
    cid = lax.axis_index("c")
    sid = lax.axis_index("s")

    _slab_copy(sid, lambda r, n: zeros_hbm.at[pl.ds(r, n)],
               lambda r, n: acc_sh.at[pl.ds(r, n)])
    plsc.subcore_barrier()

    base0 = sid * (N_EDGES // NS)

    @pl.when(cid == 0)
    def _():
      _edge_pipeline(t0_hbm, src_hbm, dst_hbm, acc_sh, base0, CHUNK1,
                     N_CHUNKS1, TAIL1, src_a, dst_a, rows_a, sem_a,
                     src_b, dst_b, rows_b, sem_b, src_t, dst_t, rows_t, sem_t)

    @pl.when(cid == 1)
    def _():
      _edge_pipeline(t1_hbm, src_hbm, dst_hbm, acc_sh, base0, CHUNK1,
                     N_CHUNKS1, TAIL1, src_a, dst_a, rows_a, sem_a,
                     src_b, dst_b, rows_b, sem_b, src_t, dst_t, rows_t, sem_t)

    plsc.subcore_barrier()

    @pl.when(cid == 0)
    def _():
      _slab_copy(sid, lambda r, n: acc_sh.at[pl.ds(r, n)],
                 lambda r, n: out0_hbm.at[pl.ds(r, n)])

    @pl.when(cid == 1)
    def _():
      _slab_copy(sid, lambda r, n: acc_sh.at[pl.ds(r, n)],
                 lambda r, n: out1_hbm.at[pl.ds(r, n)])

  return agg


_agg_l1 = _make_agg_colsplit(D_IN // 2)


CHUNK2 = 128
N_CHUNKS2 = N_EDGES // NC // NS // CHUNK2
TAIL2 = N_EDGES // NC // NS - N_CHUNKS2 * CHUNK2


def _make_agg_edgesplit(dc):
  epc = N_EDGES // NC
  eps = epc // NS
  mesh = plsc.VectorSubcoreMesh(core_axis_name="c", subcore_axis_name="s")
  out_type = (
      jax.ShapeDtypeStruct((N_NODES, dc), jnp.float32),
      jax.ShapeDtypeStruct((N_NODES, dc), jnp.float32),
  )

  @functools.partial(pl.kernel, out_type=out_type, mesh=mesh,
                     scratch_types=_gather_scratch(CHUNK2, TAIL2, dc))
  def agg(table_hbm, src_hbm, dst_hbm, zeros_hbm, out0_hbm, out1_hbm,
          src_a, dst_a, rows_a, sem_a, src_b, dst_b, rows_b, sem_b,
          src_t, dst_t, rows_t, sem_t, acc_sh):
    cid = lax.axis_index("c")
    sid = lax.axis_index("s")

    _slab_copy(sid, lambda r, n: zeros_hbm.at[pl.ds(r, n)],
               lambda r, n: acc_sh.at[pl.ds(r, n)])
    plsc.subcore_barrier()

    base0 = cid * epc + sid * eps
    _edge_pipeline(table_hbm, src_hbm, dst_hbm, acc_sh, base0, CHUNK2,
                   N_CHUNKS2, TAIL2, src_a, dst_a, rows_a, sem_a,
                   src_b, dst_b, rows_b, sem_b, src_t, dst_t, rows_t, sem_t)

    plsc.subcore_barrier()

    @pl.when(cid == 0)
    def _():
      _slab_copy(sid, lambda r, n: acc_sh.at[pl.ds(r, n)],
                 lambda r, n: out0_hbm.at[pl.ds(r, n)])

    @pl.when(cid == 1)
    def _():
      _slab_copy(sid, lambda r, n: acc_sh.at[pl.ds(r, n)],
                 lambda r, n: out1_hbm.at[pl.ds(r, n)])

  return agg


_agg_l2 = _make_agg_edgesplit(D_H2)


BM = 1000


def _mm_body(a0_ref, a1_ref, w1_ref, w2_ref, o_ref):
  h = jnp.dot(a0_ref[...], w1_ref[0:128, :], preferred_element_type=jnp.float32)
  h = h + jnp.dot(a1_ref[...], w1_ref[128:256, :],
                  preferred_element_type=jnp.float32)
  h1 = jnp.where(h > 0, h, jnp.exp(jnp.minimum(h, 0.0)) - 1.0)
  o_ref[...] = jnp.dot(h1, w2_ref[...], preferred_element_type=jnp.float32)


def _fused_mm(a0, a1, w1, w2):
  grid = (N_NODES // BM,)
  return pl.pallas_call(
      _mm_body,
      grid=grid,
      in_specs=[
          pl.BlockSpec((BM, 128), lambda i: (i, 0)),
          pl.BlockSpec((BM, 128), lambda i: (i, 0)),
          pl.BlockSpec((256, 256), lambda i: (0, 0)),
          pl.BlockSpec((256, 128), lambda i: (0, 0)),
      ],
      out_specs=pl.BlockSpec((BM, 128), lambda i: (i, 0)),
      out_shape=jax.ShapeDtypeStruct((N_NODES, D_H2), jnp.float32),
  )(a0, a1, w1, w2)


def _norm_body(p0_ref, p1_ref, o_ref):
  h2 = p0_ref[...] + p1_ref[...]
  ss = jnp.sum(h2 * h2, axis=1, keepdims=True)
  inv = 1.0 / jnp.maximum(jnp.sqrt(ss), 1e-12)
  o_ref[...] = h2 * inv


def _normalize(p0, p1):
  grid = (N_NODES // BM,)
  return pl.pallas_call(
      _norm_body,
      grid=grid,
      in_specs=[
          pl.BlockSpec((BM, 128), lambda i: (i, 0)),
          pl.BlockSpec((BM, 128), lambda i: (i, 0)),
      ],
      out_specs=pl.BlockSpec((BM, 128), lambda i: (i, 0)),
      out_shape=jax.ShapeDtypeStruct((N_NODES, D_H2), jnp.float32),
  )(p0, p1)


@jax.jit
def kernel(x, edge_index, W1, W2):
  src = edge_index[0].astype(jnp.int32)
  dst = edge_index[1].astype(jnp.int32)
  x0 = x[:, :128]
  x1 = x[:, 128:]
  z128 = jnp.zeros((N_NODES, 128), jnp.float32)

  agg0, agg1 = _agg_l1(x0, x1, src, dst, z128)
  s2 = _fused_mm(agg0, agg1, W1, W2)
  p0, p1 = _agg_l2(s2, src, dst, z128)
  return _normalize(p0, p1)


# speedup vs baseline: 9.3667x; 1.2627x over previous
import functools

import jax
import jax.numpy as jnp
from jax import lax
from jax.experimental import pallas as pl
from jax.experimental.pallas import tpu as pltpu
from jax.experimental.pallas import tpu_sc as plsc

N_NODES = 10000
N_EDGES = 160000
D_IN = 256
D_H1 = 256
D_H2 = 128

NC = 2
NS = 16
SLAB = 632
SLAB_LAST = N_NODES - (NS - 1) * SLAB


def _slab_copy(sid, make_src, make_dst):
  row0 = sid * SLAB

  @pl.when(sid < NS - 1)
  def _():
    pltpu.sync_copy(make_src(row0, SLAB), make_dst(row0, SLAB))

  @pl.when(sid == NS - 1)
  def _():
    pltpu.sync_copy(make_src(row0, SLAB_LAST), make_dst(row0, SLAB_LAST))


def _edge_prologue(table_hbm, src_hbm, dst_hbm, base0, chunk, n_chunks, tail,
                   srcs_v, dsts_v, rows_a, sem_a, rows_t, sem_t):
  n_sub = chunk * n_chunks + tail
  pltpu.sync_copy(src_hbm.at[pl.ds(base0, n_sub)], srcs_v)
  pltpu.sync_copy(dst_hbm.at[pl.ds(base0, n_sub)], dsts_v)
  if tail:
    pltpu.async_copy(
        table_hbm.at[srcs_v.at[pl.ds(n_chunks * chunk, tail)]], rows_t, sem_t)
  pltpu.async_copy(table_hbm.at[srcs_v.at[pl.ds(0, chunk)]], rows_a, sem_a)


def _edge_pipeline(table_hbm, acc_sh, chunk, n_chunks, tail,
                   srcs_v, dsts_v, rows_a, sem_a, rows_b, sem_b,
                   rows_t, sem_t):
  def src_at(c):
    return srcs_v.at[pl.ds(c * chunk, chunk)]

  def dst_at(c):
    return dsts_v.at[pl.ds(c * chunk, chunk)]

  def pair(j, carry):
    c0 = 2 * j
    pltpu.async_copy(table_hbm.at[src_at(c0 + 1)], rows_b, sem_b)
    pltpu.make_async_copy(table_hbm.at[src_at(c0)], rows_a, sem_a).wait()
    pltpu.sync_copy(rows_a, acc_sh.at[dst_at(c0)], add=True)
    pltpu.async_copy(table_hbm.at[src_at(c0 + 2)], rows_a, sem_a)
    pltpu.make_async_copy(table_hbm.at[src_at(c0 + 1)], rows_b, sem_b).wait()
    pltpu.sync_copy(rows_b, acc_sh.at[dst_at(c0 + 1)], add=True)
    return carry

  if n_chunks % 2:
    lax.fori_loop(0, (n_chunks - 1) // 2, pair, 0)
    pltpu.make_async_copy(table_hbm.at[src_at(n_chunks - 1)], rows_a,
                          sem_a).wait()
    pltpu.sync_copy(rows_a, acc_sh.at[dst_at(n_chunks - 1)], add=True)
  else:
    lax.fori_loop(0, (n_chunks - 2) // 2, pair, 0)
    pltpu.async_copy(table_hbm.at[src_at(n_chunks - 1)], rows_b, sem_b)
    pltpu.make_async_copy(table_hbm.at[src_at(n_chunks - 2)], rows_a,
                          sem_a).wait()
    pltpu.sync_copy(rows_a, acc_sh.at[dst_at(n_chunks - 2)], add=True)
    pltpu.make_async_copy(table_hbm.at[src_at(n_chunks - 1)], rows_b,
                          sem_b).wait()
    pltpu.sync_copy(rows_b, acc_sh.at[dst_at(n_chunks - 1)], add=True)

  if tail:
    pltpu.make_async_copy(
        table_hbm.at[srcs_v.at[pl.ds(n_chunks * chunk, tail)]], rows_t,
        sem_t).wait()
    pltpu.sync_copy(rows_t, acc_sh.at[dsts_v.at[pl.ds(n_chunks * chunk,
                                                      tail)]], add=True)


def _gather_scratch(chunk, n_chunks, tail, dc):
  n_sub = chunk * n_chunks + tail
  return [
      pltpu.VMEM((n_sub,), jnp.int32),
      pltpu.VMEM((n_sub,), jnp.int32),
      pltpu.VMEM((chunk, dc), jnp.float32),
      pltpu.SemaphoreType.DMA,
      pltpu.VMEM((chunk, dc), jnp.float32),
      pltpu.SemaphoreType.DMA,
      pltpu.VMEM((max(tail, 8), dc), jnp.float32),
      pltpu.SemaphoreType.DMA,
      pltpu.VMEM_SHARED((N_NODES, dc), jnp.float32),
  ]


CHUNK1 = 96
N_CHUNKS1 = N_EDGES // NS // CHUNK1
TAIL1 = N_EDGES // NS - N_CHUNKS1 * CHUNK1


def _make_agg_colsplit(dc):
  mesh = plsc.VectorSubcoreMesh(core_axis_name="c", subcore_axis_name="s")
  out_type = (
      jax.ShapeDtypeStruct((N_NODES, dc), jnp.float32),
      jax.ShapeDtypeStruct((N_NODES, dc), jnp.float32),
  )

  @functools.partial(pl.kernel, out_type=out_type, mesh=mesh,
                     scratch_types=_gather_scratch(CHUNK1, N_CHUNKS1, TAIL1,
                                                   dc))
  def agg(t0_hbm, t1_hbm, src_hbm, dst_hbm, zeros_hbm, out0_hbm, out1_hbm,
          srcs_v, dsts_v, rows_a, sem_a, rows_b, sem_b, rows_t, sem_t,
          acc_sh):
    cid = lax.axis_index("c")
    sid = lax.axis_index("s")
    base0 = sid * (N_EDGES // NS)

    @pl.when(cid == 0)
    def _():
      _edge_prologue(t0_hbm, src_hbm, dst_hbm, base0, CHUNK1, N_CHUNKS1,
                     TAIL1, srcs_v, dsts_v, rows_a, sem_a, rows_t, sem_t)

    @pl.when(cid == 1)
    def _():
      _edge_prologue(t1_hbm, src_hbm, dst_hbm, base0, CHUNK1, N_CHUNKS1,
                     TAIL1, srcs_v, dsts_v, rows_a, sem_a, rows_t, sem_t)

    _slab_copy(sid, lambda r, n: zeros_hbm.at[pl.ds(r, n)],
               lambda r, n: acc_sh.at[pl.ds(r, n)])
    plsc.subcore_barrier()

    @pl.when(cid == 0)
    def _():
      _edge_pipeline(t0_hbm, acc_sh, CHUNK1, N_CHUNKS1, TAIL1, srcs_v, dsts_v,
                     rows_a, sem_a, rows_b, sem_b, rows_t, sem_t)

    @pl.when(cid == 1)
    def _():
      _edge_pipeline(t1_hbm, acc_sh, CHUNK1, N_CHUNKS1, TAIL1, srcs_v, dsts_v,
                     rows_a, sem_a, rows_b, sem_b, rows_t, sem_t)

    plsc.subcore_barrier()

    @pl.when(cid == 0)
    def _():
      _slab_copy(sid, lambda r, n: acc_sh.at[pl.ds(r, n)],
                 lambda r, n: out0_hbm.at[pl.ds(r, n)])

    @pl.when(cid == 1)
    def _():
      _slab_copy(sid, lambda r, n: acc_sh.at[pl.ds(r, n)],
                 lambda r, n: out1_hbm.at[pl.ds(r, n)])

  return agg


_agg_l1 = _make_agg_colsplit(D_IN // 2)


CHUNK2 = 128
N_CHUNKS2 = N_EDGES // NC // NS // CHUNK2
TAIL2 = N_EDGES // NC // NS - N_CHUNKS2 * CHUNK2


def _make_agg_edgesplit(dc):
  epc = N_EDGES // NC
  eps = epc // NS
  mesh = plsc.VectorSubcoreMesh(core_axis_name="c", subcore_axis_name="s")
  out_type = (
      jax.ShapeDtypeStruct((N_NODES, dc), jnp.float32),
      jax.ShapeDtypeStruct((N_NODES, dc), jnp.float32),
  )

  @functools.partial(pl.kernel, out_type=out_type, mesh=mesh,
                     scratch_types=_gather_scratch(CHUNK2, N_CHUNKS2, TAIL2,
                                                   dc))
  def agg(table_hbm, src_hbm, dst_hbm, zeros_hbm, out0_hbm, out1_hbm,
          srcs_v, dsts_v, rows_a, sem_a, rows_b, sem_b, rows_t, sem_t,
          acc_sh):
    cid = lax.axis_index("c")
    sid = lax.axis_index("s")
    base0 = cid * epc + sid * eps

    _edge_prologue(table_hbm, src_hbm, dst_hbm, base0, CHUNK2, N_CHUNKS2,
                   TAIL2, srcs_v, dsts_v, rows_a, sem_a, rows_t, sem_t)
    _slab_copy(sid, lambda r, n: zeros_hbm.at[pl.ds(r, n)],
               lambda r, n: acc_sh.at[pl.ds(r, n)])
    plsc.subcore_barrier()

    _edge_pipeline(table_hbm, acc_sh, CHUNK2, N_CHUNKS2, TAIL2, srcs_v,
                   dsts_v, rows_a, sem_a, rows_b, sem_b, rows_t, sem_t)

    plsc.subcore_barrier()

    @pl.when(cid == 0)
    def _():
      _slab_copy(sid, lambda r, n: acc_sh.at[pl.ds(r, n)],
                 lambda r, n: out0_hbm.at[pl.ds(r, n)])

    @pl.when(cid == 1)
    def _():
      _slab_copy(sid, lambda r, n: acc_sh.at[pl.ds(r, n)],
                 lambda r, n: out1_hbm.at[pl.ds(r, n)])

  return agg


_agg_l2 = _make_agg_edgesplit(D_H2)


BM = 1000


def _mm_body(a0_ref, a1_ref, w1_ref, w2_ref, o_ref):
  h = jnp.dot(a0_ref[...], w1_ref[0:128, :], preferred_element_type=jnp.float32)
  h = h + jnp.dot(a1_ref[...], w1_ref[128:256, :],
                  preferred_element_type=jnp.float32)
  h1 = jnp.where(h > 0, h, jnp.exp(jnp.minimum(h, 0.0)) - 1.0)
  o_ref[...] = jnp.dot(h1, w2_ref[...], preferred_element_type=jnp.float32)


def _fused_mm(a0, a1, w1, w2):
  grid = (N_NODES // BM,)
  return pl.pallas_call(
      _mm_body,
      grid=grid,
      in_specs=[
          pl.BlockSpec((BM, 128), lambda i: (i, 0)),
          pl.BlockSpec((BM, 128), lambda i: (i, 0)),
          pl.BlockSpec((256, 256), lambda i: (0, 0)),
          pl.BlockSpec((256, 128), lambda i: (0, 0)),
      ],
      out_specs=pl.BlockSpec((BM, 128), lambda i: (i, 0)),
      out_shape=jax.ShapeDtypeStruct((N_NODES, D_H2), jnp.float32),
  )(a0, a1, w1, w2)


def _norm_body(p0_ref, p1_ref, o_ref):
  h2 = p0_ref[...] + p1_ref[...]
  ss = jnp.sum(h2 * h2, axis=1, keepdims=True)
  inv = 1.0 / jnp.maximum(jnp.sqrt(ss), 1e-12)
  o_ref[...] = h2 * inv


def _normalize(p0, p1):
  grid = (N_NODES // BM,)
  return pl.pallas_call(
      _norm_body,
      grid=grid,
      in_specs=[
          pl.BlockSpec((BM, 128), lambda i: (i, 0)),
          pl.BlockSpec((BM, 128), lambda i: (i, 0)),
      ],
      out_specs=pl.BlockSpec((BM, 128), lambda i: (i, 0)),
      out_shape=jax.ShapeDtypeStruct((N_NODES, D_H2), jnp.float32),
  )(p0, p1)


@jax.jit
def kernel(x, edge_index, W1, W2):
  src = edge_index[0].astype(jnp.int32)
  dst = edge_index[1].astype(jnp.int32)
  x0 = x[:, :128]
  x1 = x[:, 128:]
  z128 = jnp.zeros((N_NODES, 128), jnp.float32)

  agg0, agg1 = _agg_l1(x0, x1, src, dst, z128)
  s2 = _fused_mm(agg0, agg1, W1, W2)
  p0, p1 = _agg_l2(s2, src, dst, z128)
  return _normalize(p0, p1)
